# baseline (speedup 1.0000x reference)
import jax
import jax.numpy as jnp
from jax import lax
from jax.experimental import pallas as pl
from jax.experimental.pallas import tpu as pltpu
from jax.experimental.pallas import tpu_sc as plsc

N = 10000
E = 160000
DZ = 64

NC = 2
NS = 16
NW = NC * NS

ROWS_PER_W = 313
N_PAD = ROWS_PER_W * NW
NEG = -1.0e30

GCH = 1000
EDGES_PER_W = E // NW

DCH = 8000
NCHUNK = E // DCH
G = 128
FLUSH = 4096
MBUF = FLUSH + DCH + 256


def _zw_body(x_ref, posp_ref, w1a_ref, w1b_ref, b1_ref, z_ref, w_ref):
  pw = jnp.dot(posp_ref[...], w1b_ref[...], preferred_element_type=jnp.float32)
  z_ref[...] = (
      jnp.dot(x_ref[...], w1a_ref[...], preferred_element_type=jnp.float32)
      + pw + b1_ref[...]
  )
  w_ref[...] = pw


def _stage_a(x, posp, w1a, w1bp, b1):
  blk = 1000
  return pl.pallas_call(
      _zw_body,
      grid=(N // blk,),
      in_specs=[
          pl.BlockSpec((blk, 256), lambda i: (i, 0)),
          pl.BlockSpec((blk, 8), lambda i: (i, 0)),
          pl.BlockSpec((256, DZ), lambda i: (0, 0)),
          pl.BlockSpec((8, DZ), lambda i: (0, 0)),
          pl.BlockSpec((1, DZ), lambda i: (0, 0)),
      ],
      out_specs=[
          pl.BlockSpec((blk, DZ), lambda i: (i, 0)),
          pl.BlockSpec((blk, DZ), lambda i: (i, 0)),
      ],
      out_shape=[
          jax.ShapeDtypeStruct((N, DZ), jnp.float32),
          jax.ShapeDtypeStruct((N, DZ), jnp.float32),
      ],
      compiler_params=pltpu.CompilerParams(
          dimension_semantics=("arbitrary",)),
  )(x, posp, w1a, w1bp, b1)


def _gather_body(z_hbm, w_hbm, src_hbm, dst_hbm, zs_hbm, wd_hbm,
                 idx_v, rows_v, sem):
  wid = lax.axis_index("s") * NC + lax.axis_index("c")
  for tbl_hbm, eidx_hbm, out_hbm in ((z_hbm, src_hbm, zs_hbm),
                                     (w_hbm, dst_hbm, wd_hbm)):
    for c in range(EDGES_PER_W // GCH):
      base = wid * EDGES_PER_W + c * GCH
      pltpu.sync_copy(eidx_hbm.at[pl.ds(base, GCH)], idx_v)
      pltpu.async_copy(tbl_hbm.at[idx_v], rows_v, sem).wait()
      pltpu.sync_copy(rows_v, out_hbm.at[pl.ds(base, GCH)])


def _stage_b(z, w, src, dst):
  mesh = plsc.VectorSubcoreMesh(core_axis_name="c", subcore_axis_name="s")
  f = pl.kernel(
      _gather_body,
      out_type=[
          jax.ShapeDtypeStruct((E, DZ), jnp.float32),
          jax.ShapeDtypeStruct((E, DZ), jnp.float32),
      ],
      mesh=mesh,
      scratch_types=[
          pltpu.VMEM((GCH,), jnp.int32),
          pltpu.VMEM((GCH, DZ), jnp.float32),
          pltpu.SemaphoreType.DMA,
      ],
      compiler_params=pltpu.CompilerParams(use_tc_tiling_on_sc=False,
                                          needs_layout_passes=False),
  )
  return f(z, w, src, dst)


def _mlp_body(zs_ref, wd_ref, w2_ref, b2_ref, w3_ref, b3_ref, h3_ref):
  h1 = jnp.maximum(zs_ref[...] - wd_ref[...], 0.0).astype(jnp.bfloat16)
  h2 = jnp.maximum(
      jnp.dot(h1, w2_ref[...], preferred_element_type=jnp.float32)
      + b2_ref[...], 0.0).astype(jnp.bfloat16)
  h3_ref[...] = (
      jnp.dot(h2, w3_ref[...], preferred_element_type=jnp.float32)
      + b3_ref[...]).astype(jnp.bfloat16)


def _stage_c(zs, wd, w2, b2, w3, b3):
  blk = 2000
  return pl.pallas_call(
      _mlp_body,
      grid=(E // blk,),
      in_specs=[
          pl.BlockSpec((blk, DZ), lambda i: (i, 0)),
          pl.BlockSpec((blk, DZ), lambda i: (i, 0)),
          pl.BlockSpec((DZ, 128), lambda i: (0, 0)),
          pl.BlockSpec((1, 128), lambda i: (0, 0)),
          pl.BlockSpec((128, 256), lambda i: (0, 0)),
          pl.BlockSpec((1, 256), lambda i: (0, 0)),
      ],
      out_specs=pl.BlockSpec((blk, 256), lambda i: (i, 0)),
      out_shape=jax.ShapeDtypeStruct((E, 256), jnp.bfloat16),
      compiler_params=pltpu.CompilerParams(
          dimension_semantics=("arbitrary",)),
  )(zs, wd, w2, b2, w3, b3)


AGG_W = (ROWS_PER_W + 1) * 128


def _segmax_body(h3_hbm, dst_hbm, agg_hbm, aggf, dstbuf, meid, mld,
                 rows_a, rows_b, sem_a, sem_b):
  wid = lax.axis_index("s") * NC + lax.axis_index("c")
  lo = wid * ROWS_PER_W
  hi = lo + ROWS_PER_W
  iota = lax.iota(jnp.int32, 16)
  neg_pair = plsc.bitcast(jnp.full((32,), NEG, jnp.bfloat16), jnp.int32)

  def init_body(i, _):
    aggf[pl.ds(i * 16, 16)] = neg_pair
    return 0
  lax.fori_loop(0, AGG_W // 16, init_body, 0)

  def minit_body(i, _):
    meid[pl.ds(i * 16, 16)] = jnp.zeros((16,), jnp.int32)
    return 0
  lax.fori_loop(0, MBUF // 16, minit_body, 0)

  def issue(g, rows_ref, sem):
    pltpu.async_copy(h3_hbm.at[meid.at[pl.ds(g * G, G)]], rows_ref, sem)

  def drain(rows_ref, sem):
    pltpu.make_async_copy(h3_hbm.at[meid.at[pl.ds(0, G)]], rows_ref,
                          sem).wait()

  def flush(n):
    ng = (n + G - 1) // G

    def process(g, rows_ref):
      gb = g * G

      def row_body(r, _):
        jr = gb + r
        jr_v = jnp.full((16,), jr, jnp.int32)
        ldb = plsc.load_gather(mld, [jr_v])
        ld_safe = jnp.where(jr_v < n, ldb,
                            jnp.full((16,), ROWS_PER_W, jnp.int32))
        base = ld_safe * 128
        for k in range(8):
          idx = base + (k * 16 + iota)
          cur = plsc.bitcast(plsc.load_gather(aggf, [idx]), jnp.bfloat16)
          val = rows_ref[r, pl.ds(k * 32, 32)]
          mx = plsc.bitcast(jnp.maximum(cur, val), jnp.int32)
          plsc.store_scatter(aggf, [idx], mx)
        return 0
      lax.fori_loop(0, jnp.minimum(G, n - gb), row_body, 0)

    @pl.when(ng > 0)
    def _():
      issue(0, rows_a, sem_a)

    def pair_body(p, _):
      ga = 2 * p
      gb_ = 2 * p + 1

      @pl.when(gb_ < ng)
      def _():
        issue(gb_, rows_b, sem_b)
      drain(rows_a, sem_a)
      process(ga, rows_a)

      @pl.when(gb_ < ng)
      def _():
        @pl.when(gb_ + 1 < ng)
        def _():
          issue(gb_ + 1, rows_a, sem_a)
        drain(rows_b, sem_b)
        process(gb_, rows_b)
      return 0

    lax.fori_loop(0, (ng + 1) // 2, pair_body, 0)

  def chunk_body(c, off_vec):
    cbase = c * DCH
    pltpu.sync_copy(dst_hbm.at[pl.ds(cbase, DCH)], dstbuf)

    def comp_body(i, ov):
      d = dstbuf[pl.ds(i * 16, 16)]
      m = (d >= lo) & (d < hi)
      pos = ov + plsc.cumsum(m.astype(jnp.int32)) - 1
      eid = cbase + i * 16 + iota
      plsc.store_scatter(meid, [pos], eid, mask=m)
      plsc.store_scatter(mld, [pos], d - lo, mask=m)
      return ov + plsc.all_reduce_population_count(m)

    off_vec = lax.fori_loop(0, DCH // 16, comp_body, off_vec)
    n = jnp.max(off_vec)
    do_flush = (n >= FLUSH) | (c >= NCHUNK - 1)

    @pl.when(do_flush)
    def _():
      flush(n)

    return jnp.where(jnp.full((16,), do_flush), 0, off_vec)

  lax.fori_loop(0, NCHUNK, chunk_body, jnp.zeros((16,), jnp.int32))

  pltpu.sync_copy(aggf.at[pl.ds(0, ROWS_PER_W * 128)],
                  agg_hbm.at[pl.ds(wid * ROWS_PER_W * 128,
                                   ROWS_PER_W * 128)])


def _stage_d(h3, dst):
  mesh = plsc.VectorSubcoreMesh(core_axis_name="c", subcore_axis_name="s")
  f = pl.kernel(
      _segmax_body,
      out_type=jax.ShapeDtypeStruct((N_PAD * 128,), jnp.int32),
      mesh=mesh,
      scratch_types=[
          pltpu.VMEM((AGG_W,), jnp.int32),
          pltpu.VMEM((DCH,), jnp.int32),
          pltpu.VMEM((MBUF,), jnp.int32),
          pltpu.VMEM((MBUF,), jnp.int32),
          pltpu.VMEM((G, 256), jnp.bfloat16),
          pltpu.VMEM((G, 256), jnp.bfloat16),
          pltpu.SemaphoreType.DMA,
          pltpu.SemaphoreType.DMA,
      ],
      compiler_params=pltpu.CompilerParams(use_tc_tiling_on_sc=False,
                                          needs_layout_passes=False),
  )
  return f(h3, dst)


def _out_body(agg_ref, wg_ref, bg_ref, out_ref):
  a = agg_ref[...]
  a = jnp.where(a > jnp.bfloat16(-1.0e29), a, jnp.bfloat16(0.0))
  out_ref[...] = jnp.maximum(
      jnp.dot(a, wg_ref[...], preferred_element_type=jnp.float32)
      + bg_ref[...], 0.0)


def _stage_e(agg, wg, bg):
  blk = 1000
  return pl.pallas_call(
      _out_body,
      grid=(N // blk,),
      in_specs=[
          pl.BlockSpec((blk, 256), lambda i: (i, 0)),
          pl.BlockSpec((256, 256), lambda i: (0, 0)),
          pl.BlockSpec((1, 256), lambda i: (0, 0)),
      ],
      out_specs=pl.BlockSpec((blk, 256), lambda i: (i, 0)),
      out_shape=jax.ShapeDtypeStruct((N, 256), jnp.float32),
      compiler_params=pltpu.CompilerParams(
          dimension_semantics=("arbitrary",)),
  )(agg, wg, bg)


def kernel(x, pos, edge_index, W1, b1, W2, b2, W3, b3, Wg, bg):
  src = edge_index[0].astype(jnp.int32)
  dst = edge_index[1].astype(jnp.int32)
  w1a = W1[:256]
  w1bp = jnp.zeros((8, DZ), jnp.float32).at[:3].set(W1[256:])
  posp = jnp.zeros((N, 8), jnp.float32).at[:, :3].set(pos)

  z, w = _stage_a(x, posp, w1a, w1bp, b1.reshape(1, DZ))
  zs, wd = _stage_b(z, w, src, dst)
  h3 = _stage_c(zs, wd, W2.astype(jnp.bfloat16), b2.reshape(1, 128),
                W3.astype(jnp.bfloat16), b3.reshape(1, 256))
  agg1d = _stage_d(h3, dst)
  agg = jax.lax.bitcast_convert_type(agg1d, jnp.bfloat16).reshape(
      N_PAD, 256)[:N]
  return _stage_e(agg, Wg.astype(jnp.bfloat16), bg.reshape(1, 256))


# speedup vs baseline: 1.6415x; 1.6415x over previous
import jax
import jax.numpy as jnp
from jax import lax
from jax.experimental import pallas as pl
from jax.experimental.pallas import tpu as pltpu
from jax.experimental.pallas import tpu_sc as plsc

N = 10000
E = 160000
DZ = 64

NC = 2
NS = 16
NW = NC * NS

ROWS_PER_W = 313
N_PAD = ROWS_PER_W * NW
NEG = -1.0e30

GCH = 1000
EDGES_PER_W = E // NW

DCH = 8000
NCHUNK = E // DCH
G = 128
FLUSH = 4096
MBUF = FLUSH + DCH + 256


def _zw_body(x_ref, posp_ref, w1a_ref, w1b_ref, b1_ref, z_ref, w_ref):
  pw = jnp.dot(posp_ref[...], w1b_ref[...], preferred_element_type=jnp.float32)
  z_ref[...] = (
      jnp.dot(x_ref[...], w1a_ref[...], preferred_element_type=jnp.float32)
      + pw + b1_ref[...]
  )
  w_ref[...] = pw


def _stage_a(x, posp, w1a, w1bp, b1):
  blk = 1000
  return pl.pallas_call(
      _zw_body,
      grid=(N // blk,),
      in_specs=[
          pl.BlockSpec((blk, 256), lambda i: (i, 0)),
          pl.BlockSpec((blk, 8), lambda i: (i, 0)),
          pl.BlockSpec((256, DZ), lambda i: (0, 0)),
          pl.BlockSpec((8, DZ), lambda i: (0, 0)),
          pl.BlockSpec((1, DZ), lambda i: (0, 0)),
      ],
      out_specs=[
          pl.BlockSpec((blk, DZ), lambda i: (i, 0)),
          pl.BlockSpec((blk, DZ), lambda i: (i, 0)),
      ],
      out_shape=[
          jax.ShapeDtypeStruct((N, DZ), jnp.float32),
          jax.ShapeDtypeStruct((N, DZ), jnp.float32),
      ],
      compiler_params=pltpu.CompilerParams(
          dimension_semantics=("arbitrary",)),
  )(x, posp, w1a, w1bp, b1)


def _gather_body(z_hbm, w_hbm, src_hbm, dst_hbm, zs_hbm, wd_hbm,
                 idx_v, rows_v, sem):
  wid = lax.axis_index("s") * NC + lax.axis_index("c")
  for tbl_hbm, eidx_hbm, out_hbm in ((z_hbm, src_hbm, zs_hbm),
                                     (w_hbm, dst_hbm, wd_hbm)):
    for c in range(EDGES_PER_W // GCH):
      base = wid * EDGES_PER_W + c * GCH
      pltpu.sync_copy(eidx_hbm.at[pl.ds(base, GCH)], idx_v)
      pltpu.async_copy(tbl_hbm.at[idx_v], rows_v, sem).wait()
      pltpu.sync_copy(rows_v, out_hbm.at[pl.ds(base, GCH)])


def _stage_b(z, w, src, dst):
  mesh = plsc.VectorSubcoreMesh(core_axis_name="c", subcore_axis_name="s")
  f = pl.kernel(
      _gather_body,
      out_type=[
          jax.ShapeDtypeStruct((E, DZ), jnp.float32),
          jax.ShapeDtypeStruct((E, DZ), jnp.float32),
      ],
      mesh=mesh,
      scratch_types=[
          pltpu.VMEM((GCH,), jnp.int32),
          pltpu.VMEM((GCH, DZ), jnp.float32),
          pltpu.SemaphoreType.DMA,
      ],
      compiler_params=pltpu.CompilerParams(use_tc_tiling_on_sc=False,
                                          needs_layout_passes=False),
  )
  return f(z, w, src, dst)


def _mlp_body(zs_ref, wd_ref, w2_ref, b2_ref, w3_ref, b3_ref, h3_ref):
  h1 = jnp.maximum(zs_ref[...] - wd_ref[...], 0.0).astype(jnp.bfloat16)
  h2 = jnp.maximum(
      jnp.dot(h1, w2_ref[...], preferred_element_type=jnp.float32)
      + b2_ref[...], 0.0).astype(jnp.bfloat16)
  h3_ref[...] = (
      jnp.dot(h2, w3_ref[...], preferred_element_type=jnp.float32)
      + b3_ref[...]).astype(jnp.bfloat16)


def _stage_c(zs, wd, w2, b2, w3, b3):
  blk = 2000
  return pl.pallas_call(
      _mlp_body,
      grid=(E // blk,),
      in_specs=[
          pl.BlockSpec((blk, DZ), lambda i: (i, 0)),
          pl.BlockSpec((blk, DZ), lambda i: (i, 0)),
          pl.BlockSpec((DZ, 128), lambda i: (0, 0)),
          pl.BlockSpec((1, 128), lambda i: (0, 0)),
          pl.BlockSpec((128, 256), lambda i: (0, 0)),
          pl.BlockSpec((1, 256), lambda i: (0, 0)),
      ],
      out_specs=pl.BlockSpec((blk, 256), lambda i: (i, 0)),
      out_shape=jax.ShapeDtypeStruct((E, 256), jnp.bfloat16),
      compiler_params=pltpu.CompilerParams(
          dimension_semantics=("arbitrary",)),
  )(zs, wd, w2, b2, w3, b3)


AGG_W = (ROWS_PER_W + 1) * 128


def _segmax_body(h3_hbm, dst_hbm, agg_hbm, aggf, dstbuf, meid, mld,
                 rows_a, rows_b, sem_a, sem_b):
  wid = lax.axis_index("s") * NC + lax.axis_index("c")
  lo = wid * ROWS_PER_W
  hi = lo + ROWS_PER_W
  iota = lax.iota(jnp.int32, 16)
  neg_pair = plsc.bitcast(jnp.full((32,), NEG, jnp.bfloat16), jnp.int32)

  def init_body(i, _):
    aggf[pl.ds(i * 16, 16)] = neg_pair
    return 0
  lax.fori_loop(0, AGG_W // 16, init_body, 0)

  def minit_body(i, _):
    meid[pl.ds(i * 16, 16)] = jnp.zeros((16,), jnp.int32)
    return 0
  lax.fori_loop(0, MBUF // 16, minit_body, 0)

  def issue(g, rows_ref, sem):
    pltpu.async_copy(h3_hbm.at[meid.at[pl.ds(g * G, G)]], rows_ref, sem)

  def drain(rows_ref, sem):
    pltpu.make_async_copy(h3_hbm.at[meid.at[pl.ds(0, G)]], rows_ref,
                          sem).wait()

  def flush(n):
    ng = (n + G - 1) // G

    def process(g, rows_ref):
      gb = g * G

      def row_body(r, _):
        jr = gb + r
        jr_v = jnp.full((16,), jr, jnp.int32)
        ldb = plsc.load_gather(mld, [jr_v])
        ld_safe = jnp.where(jr_v < n, ldb,
                            jnp.full((16,), ROWS_PER_W, jnp.int32))
        base = ld_safe * 128
        for k in range(8):
          idx = base + (k * 16 + iota)
          cur = plsc.bitcast(plsc.load_gather(aggf, [idx]), jnp.bfloat16)
          val = rows_ref[r, pl.ds(k * 32, 32)]
          mx = plsc.bitcast(jnp.maximum(cur, val), jnp.int32)
          plsc.store_scatter(aggf, [idx], mx)
        return 0
      lax.fori_loop(0, jnp.minimum(G, n - gb), row_body, 0)

    @pl.when(ng > 0)
    def _():
      issue(0, rows_a, sem_a)

    def pair_body(p, _):
      ga = 2 * p
      gb_ = 2 * p + 1

      @pl.when(gb_ < ng)
      def _():
        issue(gb_, rows_b, sem_b)
      drain(rows_a, sem_a)
      process(ga, rows_a)

      @pl.when(gb_ < ng)
      def _():
        @pl.when(gb_ + 1 < ng)
        def _():
          issue(gb_ + 1, rows_a, sem_a)
        drain(rows_b, sem_b)
        process(gb_, rows_b)
      return 0

    lax.fori_loop(0, (ng + 1) // 2, pair_body, 0)

  def chunk_body(c, off_vec):
    cbase = c * DCH
    pltpu.sync_copy(dst_hbm.at[pl.ds(cbase, DCH)], dstbuf)

    def comp_body(i, ov):
      d = dstbuf[pl.ds(i * 16, 16)]
      m = (d >= lo) & (d < hi)
      pos = ov + plsc.cumsum(m.astype(jnp.int32)) - 1
      eid = cbase + i * 16 + iota
      plsc.store_scatter(meid, [pos], eid, mask=m)
      plsc.store_scatter(mld, [pos], d - lo, mask=m)
      return ov + plsc.all_reduce_population_count(m)

    off_vec = lax.fori_loop(0, DCH // 16, comp_body, off_vec)
    n = jnp.max(off_vec)
    do_flush = (n >= FLUSH) | (c >= NCHUNK - 1)

    @pl.when(do_flush)
    def _():
      flush(n)

    return jnp.where(jnp.full((16,), do_flush), 0, off_vec)

  lax.fori_loop(0, NCHUNK, chunk_body, jnp.zeros((16,), jnp.int32))

  pltpu.sync_copy(aggf.at[pl.ds(0, ROWS_PER_W * 128)],
                  agg_hbm.at[pl.ds(wid * ROWS_PER_W * 128,
                                   ROWS_PER_W * 128)])


def _stage_d(h3, dst):
  mesh = plsc.VectorSubcoreMesh(core_axis_name="c", subcore_axis_name="s")
  f = pl.kernel(
      _segmax_body,
      out_type=jax.ShapeDtypeStruct((N_PAD * 128,), jnp.int32),
      mesh=mesh,
      scratch_types=[
          pltpu.VMEM((AGG_W,), jnp.int32),
          pltpu.VMEM((DCH,), jnp.int32),
          pltpu.VMEM((MBUF,), jnp.int32),
          pltpu.VMEM((MBUF,), jnp.int32),
          pltpu.VMEM((G, 256), jnp.bfloat16),
          pltpu.VMEM((G, 256), jnp.bfloat16),
          pltpu.SemaphoreType.DMA,
          pltpu.SemaphoreType.DMA,
      ],
      compiler_params=pltpu.CompilerParams(use_tc_tiling_on_sc=False,
                                          needs_layout_passes=False),
  )
  return f(h3, dst)


def _out_body(aggi_ref, wge_ref, wgo_ref, bg_ref, out_ref):
  v = aggi_ref[...]
  ae = jax.lax.bitcast_convert_type(v << 16, jnp.float32)
  ao = jax.lax.bitcast_convert_type(v & jnp.int32(-65536), jnp.float32)
  ae = jnp.where(ae > -1.0e29, ae, 0.0).astype(jnp.bfloat16)
  ao = jnp.where(ao > -1.0e29, ao, 0.0).astype(jnp.bfloat16)
  out_ref[...] = jnp.maximum(
      jnp.dot(ae, wge_ref[...], preferred_element_type=jnp.float32)
      + jnp.dot(ao, wgo_ref[...], preferred_element_type=jnp.float32)
      + bg_ref[...], 0.0)


def _stage_e(aggi, wge, wgo, bg):
  blk = 1000
  return pl.pallas_call(
      _out_body,
      grid=(N // blk,),
      in_specs=[
          pl.BlockSpec((blk, 128), lambda i: (i, 0)),
          pl.BlockSpec((128, 256), lambda i: (0, 0)),
          pl.BlockSpec((128, 256), lambda i: (0, 0)),
          pl.BlockSpec((1, 256), lambda i: (0, 0)),
      ],
      out_specs=pl.BlockSpec((blk, 256), lambda i: (i, 0)),
      out_shape=jax.ShapeDtypeStruct((N, 256), jnp.float32),
      compiler_params=pltpu.CompilerParams(
          dimension_semantics=("arbitrary",)),
  )(aggi, wge, wgo, bg)


def kernel(x, pos, edge_index, W1, b1, W2, b2, W3, b3, Wg, bg):
  src = edge_index[0].astype(jnp.int32)
  dst = edge_index[1].astype(jnp.int32)
  w1a = W1[:256]
  w1bp = jnp.zeros((8, DZ), jnp.float32).at[:3].set(W1[256:])
  posp = jnp.zeros((N, 8), jnp.float32).at[:, :3].set(pos)

  z, w = _stage_a(x, posp, w1a, w1bp, b1.reshape(1, DZ))
  zs, wd = _stage_b(z, w, src, dst)
  h3 = _stage_c(zs, wd, W2.astype(jnp.bfloat16), b2.reshape(1, 128),
                W3.astype(jnp.bfloat16), b3.reshape(1, 256))
  agg1d = _stage_d(h3, dst)
  aggi = agg1d.reshape(N_PAD, 128)
  return _stage_e(aggi, Wg[0::2].astype(jnp.bfloat16),
                  Wg[1::2].astype(jnp.bfloat16), bg.reshape(1, 256))


# speedup vs baseline: 1.9050x; 1.1605x over previous
import jax
import jax.numpy as jnp
from jax import lax
from jax.experimental import pallas as pl
from jax.experimental.pallas import tpu as pltpu
from jax.experimental.pallas import tpu_sc as plsc

N = 10000
E = 160000
DZ = 64

NC = 2
NS = 16
NW = NC * NS

ROWS_PER_W = 313
N_PAD = ROWS_PER_W * NW
NEG = -1.0e30

GCH = 1000
EDGES_PER_W = E // NW

DCH = 8000
NCHUNK = E // DCH
G = 128
FLUSH = 4096
MBUF = FLUSH + DCH + 256


def _zw_body(x_ref, posp_ref, w1a_ref, w1b_ref, b1_ref, z_ref, w_ref):
  pw = jnp.dot(posp_ref[...], w1b_ref[...], preferred_element_type=jnp.float32)
  z_ref[...] = (
      jnp.dot(x_ref[...], w1a_ref[...], preferred_element_type=jnp.float32)
      + pw + b1_ref[...]
  )
  w_ref[...] = pw


def _stage_a(x, posp, w1a, w1bp, b1):
  blk = 1000
  return pl.pallas_call(
      _zw_body,
      grid=(N // blk,),
      in_specs=[
          pl.BlockSpec((blk, 256), lambda i: (i, 0)),
          pl.BlockSpec((blk, 8), lambda i: (i, 0)),
          pl.BlockSpec((256, DZ), lambda i: (0, 0)),
          pl.BlockSpec((8, DZ), lambda i: (0, 0)),
          pl.BlockSpec((1, DZ), lambda i: (0, 0)),
      ],
      out_specs=[
          pl.BlockSpec((blk, DZ), lambda i: (i, 0)),
          pl.BlockSpec((blk, DZ), lambda i: (i, 0)),
      ],
      out_shape=[
          jax.ShapeDtypeStruct((N, DZ), jnp.float32),
          jax.ShapeDtypeStruct((N, DZ), jnp.float32),
      ],
      compiler_params=pltpu.CompilerParams(
          dimension_semantics=("arbitrary",)),
  )(x, posp, w1a, w1bp, b1)


def _gather_body(z_hbm, w_hbm, src_hbm, dst_hbm, zs_hbm, wd_hbm,
                 idx_v, rows_v, sem):
  wid = lax.axis_index("s") * NC + lax.axis_index("c")
  for tbl_hbm, eidx_hbm, out_hbm in ((z_hbm, src_hbm, zs_hbm),
                                     (w_hbm, dst_hbm, wd_hbm)):
    for c in range(EDGES_PER_W // GCH):
      base = wid * EDGES_PER_W + c * GCH
      pltpu.sync_copy(eidx_hbm.at[pl.ds(base, GCH)], idx_v)
      pltpu.async_copy(tbl_hbm.at[idx_v], rows_v, sem).wait()
      pltpu.sync_copy(rows_v, out_hbm.at[pl.ds(base, GCH)])


def _stage_b(z, w, src, dst):
  mesh = plsc.VectorSubcoreMesh(core_axis_name="c", subcore_axis_name="s")
  f = pl.kernel(
      _gather_body,
      out_type=[
          jax.ShapeDtypeStruct((E, DZ), jnp.float32),
          jax.ShapeDtypeStruct((E, DZ), jnp.float32),
      ],
      mesh=mesh,
      scratch_types=[
          pltpu.VMEM((GCH,), jnp.int32),
          pltpu.VMEM((GCH, DZ), jnp.float32),
          pltpu.SemaphoreType.DMA,
      ],
      compiler_params=pltpu.CompilerParams(use_tc_tiling_on_sc=False,
                                          needs_layout_passes=False),
  )
  return f(z, w, src, dst)


def _mlp_body(zs_ref, wd_ref, w2_ref, b2_ref, w3e_ref, b3e_ref, w3o_ref,
              b3o_ref, h3_ref):
  h1 = jnp.maximum(zs_ref[...] - wd_ref[...], 0.0).astype(jnp.bfloat16)
  h2 = jnp.maximum(
      jnp.dot(h1, w2_ref[...], preferred_element_type=jnp.float32)
      + b2_ref[...], 0.0).astype(jnp.bfloat16)
  he = (jnp.dot(h2, w3e_ref[...], preferred_element_type=jnp.float32)
        + b3e_ref[...]).astype(jnp.bfloat16)
  ho = (jnp.dot(h2, w3o_ref[...], preferred_element_type=jnp.float32)
        + b3o_ref[...]).astype(jnp.bfloat16)
  lo = jax.lax.bitcast_convert_type(he, jnp.int16).astype(jnp.int32) & 0xFFFF
  hi = jax.lax.bitcast_convert_type(ho, jnp.int16).astype(jnp.int32)
  h3_ref[...] = lo | (hi << 16)


def _stage_c(zs, wd, w2, b2, w3e, b3e, w3o, b3o):
  blk = 2000
  return pl.pallas_call(
      _mlp_body,
      grid=(E // blk,),
      in_specs=[
          pl.BlockSpec((blk, DZ), lambda i: (i, 0)),
          pl.BlockSpec((blk, DZ), lambda i: (i, 0)),
          pl.BlockSpec((DZ, 128), lambda i: (0, 0)),
          pl.BlockSpec((1, 128), lambda i: (0, 0)),
          pl.BlockSpec((128, 128), lambda i: (0, 0)),
          pl.BlockSpec((1, 128), lambda i: (0, 0)),
          pl.BlockSpec((128, 128), lambda i: (0, 0)),
          pl.BlockSpec((1, 128), lambda i: (0, 0)),
      ],
      out_specs=pl.BlockSpec((blk, 128), lambda i: (i, 0)),
      out_shape=jax.ShapeDtypeStruct((E, 128), jnp.int32),
      compiler_params=pltpu.CompilerParams(
          dimension_semantics=("arbitrary",)),
  )(zs, wd, w2, b2, w3e, b3e, w3o, b3o)


AGG_W = (ROWS_PER_W + 1) * 128


def _segmax_body(h3_hbm, dst_hbm, agg_hbm, aggf, dstbuf, meid, mld,
                 rows_a, rows_b, sem_a, sem_b):
  wid = lax.axis_index("s") * NC + lax.axis_index("c")
  lo = wid * ROWS_PER_W
  hi = lo + ROWS_PER_W
  iota = lax.iota(jnp.int32, 16)
  neg_pair = plsc.bitcast(jnp.full((32,), NEG, jnp.bfloat16), jnp.int32)

  def init_body(i, _):
    aggf[pl.ds(i * 16, 16)] = neg_pair
    return 0
  lax.fori_loop(0, AGG_W // 16, init_body, 0)

  def minit_body(i, _):
    meid[pl.ds(i * 16, 16)] = jnp.zeros((16,), jnp.int32)
    return 0
  lax.fori_loop(0, MBUF // 16, minit_body, 0)

  def issue(g, rows_ref, sem):
    pltpu.async_copy(h3_hbm.at[meid.at[pl.ds(g * G, G)]], rows_ref, sem)

  def drain(rows_ref, sem):
    pltpu.make_async_copy(h3_hbm.at[meid.at[pl.ds(0, G)]], rows_ref,
                          sem).wait()

  def flush(n):
    ng = (n + G - 1) // G

    def process(g, rows_ref):
      gb = g * G

      def row_body(r, _):
        jr = gb + r
        jr_v = jnp.full((16,), jr, jnp.int32)
        ldb = plsc.load_gather(mld, [jr_v])
        ld_safe = jnp.where(jr_v < n, ldb,
                            jnp.full((16,), ROWS_PER_W, jnp.int32))
        base = ld_safe * 128
        for k in range(8):
          idx = base + (k * 16 + iota)
          cur = plsc.bitcast(plsc.load_gather(aggf, [idx]), jnp.bfloat16)
          val = plsc.bitcast(rows_ref[r, pl.ds(k * 16, 16)], jnp.bfloat16)
          mx = plsc.bitcast(jnp.maximum(cur, val), jnp.int32)
          plsc.store_scatter(aggf, [idx], mx)
        return 0
      lax.fori_loop(0, jnp.minimum(G, n - gb), row_body, 0)

    @pl.when(ng > 0)
    def _():
      issue(0, rows_a, sem_a)

    def pair_body(p, _):
      ga = 2 * p
      gb_ = 2 * p + 1

      @pl.when(gb_ < ng)
      def _():
        issue(gb_, rows_b, sem_b)
      drain(rows_a, sem_a)
      process(ga, rows_a)

      @pl.when(gb_ < ng)
      def _():
        @pl.when(gb_ + 1 < ng)
        def _():
          issue(gb_ + 1, rows_a, sem_a)
        drain(rows_b, sem_b)
        process(gb_, rows_b)
      return 0

    lax.fori_loop(0, (ng + 1) // 2, pair_body, 0)

  def chunk_body(c, off_vec):
    cbase = c * DCH
    pltpu.sync_copy(dst_hbm.at[pl.ds(cbase, DCH)], dstbuf)

    def comp_body(i, ov):
      d = dstbuf[pl.ds(i * 16, 16)]
      m = (d >= lo) & (d < hi)
      pos = ov + plsc.cumsum(m.astype(jnp.int32)) - 1
      eid = cbase + i * 16 + iota
      plsc.store_scatter(meid, [pos], eid, mask=m)
      plsc.store_scatter(mld, [pos], d - lo, mask=m)
      return ov + plsc.all_reduce_population_count(m)

    off_vec = lax.fori_loop(0, DCH // 16, comp_body, off_vec)
    n = jnp.max(off_vec)
    do_flush = (n >= FLUSH) | (c >= NCHUNK - 1)

    @pl.when(do_flush)
    def _():
      flush(n)

    return jnp.where(jnp.full((16,), do_flush), 0, off_vec)

  lax.fori_loop(0, NCHUNK, chunk_body, jnp.zeros((16,), jnp.int32))

  pltpu.sync_copy(aggf.at[pl.ds(0, ROWS_PER_W * 128)],
                  agg_hbm.at[pl.ds(wid * ROWS_PER_W * 128,
                                   ROWS_PER_W * 128)])


def _stage_d(h3, dst):
  mesh = plsc.VectorSubcoreMesh(core_axis_name="c", subcore_axis_name="s")
  f = pl.kernel(
      _segmax_body,
      out_type=jax.ShapeDtypeStruct((N_PAD * 128,), jnp.int32),
      mesh=mesh,
      scratch_types=[
          pltpu.VMEM((AGG_W,), jnp.int32),
          pltpu.VMEM((DCH,), jnp.int32),
          pltpu.VMEM((MBUF,), jnp.int32),
          pltpu.VMEM((MBUF,), jnp.int32),
          pltpu.VMEM((G, 128), jnp.int32),
          pltpu.VMEM((G, 128), jnp.int32),
          pltpu.SemaphoreType.DMA,
          pltpu.SemaphoreType.DMA,
      ],
      compiler_params=pltpu.CompilerParams(needs_layout_passes=False),
  )
  return f(h3, dst)


def _out_body(aggi_ref, wge_ref, wgo_ref, bg_ref, out_ref):
  v = aggi_ref[...]
  ae = jax.lax.bitcast_convert_type(v << 16, jnp.float32)
  ao = jax.lax.bitcast_convert_type(v & jnp.int32(-65536), jnp.float32)
  ae = jnp.where(ae > -1.0e29, ae, 0.0).astype(jnp.bfloat16)
  ao = jnp.where(ao > -1.0e29, ao, 0.0).astype(jnp.bfloat16)
  out_ref[...] = jnp.maximum(
      jnp.dot(ae, wge_ref[...], preferred_element_type=jnp.float32)
      + jnp.dot(ao, wgo_ref[...], preferred_element_type=jnp.float32)
      + bg_ref[...], 0.0)


def _stage_e(aggi, wge, wgo, bg):
  blk = 1000
  return pl.pallas_call(
      _out_body,
      grid=(N // blk,),
      in_specs=[
          pl.BlockSpec((blk, 128), lambda i: (i, 0)),
          pl.BlockSpec((128, 256), lambda i: (0, 0)),
          pl.BlockSpec((128, 256), lambda i: (0, 0)),
          pl.BlockSpec((1, 256), lambda i: (0, 0)),
      ],
      out_specs=pl.BlockSpec((blk, 256), lambda i: (i, 0)),
      out_shape=jax.ShapeDtypeStruct((N, 256), jnp.float32),
      compiler_params=pltpu.CompilerParams(
          dimension_semantics=("arbitrary",)),
  )(aggi, wge, wgo, bg)


def kernel(x, pos, edge_index, W1, b1, W2, b2, W3, b3, Wg, bg):
  src = edge_index[0].astype(jnp.int32)
  dst = edge_index[1].astype(jnp.int32)
  w1a = W1[:256]
  w1bp = jnp.zeros((8, DZ), jnp.float32).at[:3].set(W1[256:])
  posp = jnp.zeros((N, 8), jnp.float32).at[:, :3].set(pos)

  z, w = _stage_a(x, posp, w1a, w1bp, b1.reshape(1, DZ))
  zs, wd = _stage_b(z, w, src, dst)
  w3b = W3.astype(jnp.bfloat16)
  h3 = _stage_c(zs, wd, W2.astype(jnp.bfloat16), b2.reshape(1, 128),
                w3b[:, 0::2], b3[0::2].reshape(1, 128),
                w3b[:, 1::2], b3[1::2].reshape(1, 128))
  agg1d = _stage_d(h3, dst)
  aggi = agg1d.reshape(N_PAD, 128)
  return _stage_e(aggi, Wg[0::2].astype(jnp.bfloat16),
                  Wg[1::2].astype(jnp.bfloat16), bg.reshape(1, 256))


# speedup vs baseline: 1.9063x; 1.0007x over previous
import jax
import jax.numpy as jnp
from jax import lax
from jax.experimental import pallas as pl
from jax.experimental.pallas import tpu as pltpu
from jax.experimental.pallas import tpu_sc as plsc

N = 10000
E = 160000
DZ = 64

NC = 2
NS = 16
NW = NC * NS

ROWS_PER_W = 313
N_PAD = ROWS_PER_W * NW
NEG = -1.0e30

GCH = 1000
EDGES_PER_W = E // NW

DCH = 8000
NCHUNK = E // DCH
G = 128
FLUSH = 4096
MBUF = FLUSH + DCH + 256


def _zw_body(x_ref, posp_ref, w1a_ref, w1b_ref, b1_ref, z_ref, w_ref):
  pw = jnp.dot(posp_ref[...], w1b_ref[...], preferred_element_type=jnp.float32)
  z_ref[...] = (
      jnp.dot(x_ref[...], w1a_ref[...], preferred_element_type=jnp.float32)
      + pw + b1_ref[...]
  )
  w_ref[...] = pw


def _stage_a(x, posp, w1a, w1bp, b1):
  blk = 1000
  return pl.pallas_call(
      _zw_body,
      grid=(N // blk,),
      in_specs=[
          pl.BlockSpec((blk, 256), lambda i: (i, 0)),
          pl.BlockSpec((blk, 8), lambda i: (i, 0)),
          pl.BlockSpec((256, DZ), lambda i: (0, 0)),
          pl.BlockSpec((8, DZ), lambda i: (0, 0)),
          pl.BlockSpec((1, DZ), lambda i: (0, 0)),
      ],
      out_specs=[
          pl.BlockSpec((blk, DZ), lambda i: (i, 0)),
          pl.BlockSpec((blk, DZ), lambda i: (i, 0)),
      ],
      out_shape=[
          jax.ShapeDtypeStruct((N, DZ), jnp.float32),
          jax.ShapeDtypeStruct((N, DZ), jnp.float32),
      ],
      compiler_params=pltpu.CompilerParams(
          dimension_semantics=("arbitrary",)),
  )(x, posp, w1a, w1bp, b1)


def _gather_body(z_hbm, w_hbm, src_hbm, dst_hbm, zs_hbm, wd_hbm,
                 idx_v, rows_v, sem):
  wid = lax.axis_index("s") * NC + lax.axis_index("c")
  for tbl_hbm, eidx_hbm, out_hbm in ((z_hbm, src_hbm, zs_hbm),
                                     (w_hbm, dst_hbm, wd_hbm)):
    for c in range(EDGES_PER_W // GCH):
      base = wid * EDGES_PER_W + c * GCH
      pltpu.sync_copy(eidx_hbm.at[pl.ds(base, GCH)], idx_v)
      pltpu.async_copy(tbl_hbm.at[idx_v], rows_v, sem).wait()
      pltpu.sync_copy(rows_v, out_hbm.at[pl.ds(base, GCH)])


def _stage_b(z, w, src, dst):
  mesh = plsc.VectorSubcoreMesh(core_axis_name="c", subcore_axis_name="s")
  f = pl.kernel(
      _gather_body,
      out_type=[
          jax.ShapeDtypeStruct((E, DZ), jnp.float32),
          jax.ShapeDtypeStruct((E, DZ), jnp.float32),
      ],
      mesh=mesh,
      scratch_types=[
          pltpu.VMEM((GCH,), jnp.int32),
          pltpu.VMEM((GCH, DZ), jnp.float32),
          pltpu.SemaphoreType.DMA,
      ],
      compiler_params=pltpu.CompilerParams(use_tc_tiling_on_sc=False,
                                          needs_layout_passes=False),
  )
  return f(z, w, src, dst)


def _mlp_body(zs_ref, wd_ref, w2_ref, b2_ref, w3e_ref, b3e_ref, w3o_ref,
              b3o_ref, h3_ref):
  h1 = jnp.maximum(zs_ref[...] - wd_ref[...], 0.0).astype(jnp.bfloat16)
  h2 = jnp.maximum(
      jnp.dot(h1, w2_ref[...], preferred_element_type=jnp.float32)
      + b2_ref[...], 0.0).astype(jnp.bfloat16)
  he = (jnp.dot(h2, w3e_ref[...], preferred_element_type=jnp.float32)
        + b3e_ref[...]).astype(jnp.bfloat16)
  ho = (jnp.dot(h2, w3o_ref[...], preferred_element_type=jnp.float32)
        + b3o_ref[...]).astype(jnp.bfloat16)
  lo = jax.lax.bitcast_convert_type(he, jnp.int16).astype(jnp.int32) & 0xFFFF
  hi = jax.lax.bitcast_convert_type(ho, jnp.int16).astype(jnp.int32)
  h3_ref[...] = lo | (hi << 16)


def _stage_c(zs, wd, w2, b2, w3e, b3e, w3o, b3o):
  blk = 2000
  return pl.pallas_call(
      _mlp_body,
      grid=(E // blk,),
      in_specs=[
          pl.BlockSpec((blk, DZ), lambda i: (i, 0)),
          pl.BlockSpec((blk, DZ), lambda i: (i, 0)),
          pl.BlockSpec((DZ, 128), lambda i: (0, 0)),
          pl.BlockSpec((1, 128), lambda i: (0, 0)),
          pl.BlockSpec((128, 128), lambda i: (0, 0)),
          pl.BlockSpec((1, 128), lambda i: (0, 0)),
          pl.BlockSpec((128, 128), lambda i: (0, 0)),
          pl.BlockSpec((1, 128), lambda i: (0, 0)),
      ],
      out_specs=pl.BlockSpec((blk, 128), lambda i: (i, 0)),
      out_shape=jax.ShapeDtypeStruct((E, 128), jnp.int32),
      compiler_params=pltpu.CompilerParams(
          dimension_semantics=("arbitrary",)),
  )(zs, wd, w2, b2, w3e, b3e, w3o, b3o)


AGG_W = (ROWS_PER_W + 1) * 128


def _segmax_body(h3_hbm, dst_hbm, agg_hbm, aggf, dstbuf, meid, mld,
                 rows_a, rows_b, sem_a, sem_b):
  wid = lax.axis_index("s") * NC + lax.axis_index("c")
  lo = wid * ROWS_PER_W
  hi = lo + ROWS_PER_W
  iota = lax.iota(jnp.int32, 16)
  neg_pair = plsc.bitcast(jnp.full((32,), NEG, jnp.bfloat16), jnp.int32)

  def init_body(i, _):
    aggf[pl.ds(i * 16, 16)] = neg_pair
    return 0
  lax.fori_loop(0, AGG_W // 16, init_body, 0)

  def minit_body(i, _):
    meid[pl.ds(i * 16, 16)] = jnp.zeros((16,), jnp.int32)
    return 0
  lax.fori_loop(0, MBUF // 16, minit_body, 0)

  def issue(g, rows_ref, sem):
    pltpu.async_copy(h3_hbm.at[meid.at[pl.ds(g * G, G)]], rows_ref, sem)

  def drain(rows_ref, sem):
    pltpu.make_async_copy(h3_hbm.at[meid.at[pl.ds(0, G)]], rows_ref,
                          sem).wait()

  def flush(n):
    ng = (n + G - 1) // G

    def process(g, rows_ref):
      gb = g * G

      def row_body(r, _):
        jr = gb + r
        jr_v = jnp.full((16,), jr, jnp.int32)
        ldb = plsc.load_gather(mld, [jr_v])
        ld_safe = jnp.where(jr_v < n, ldb,
                            jnp.full((16,), ROWS_PER_W, jnp.int32))
        base = ld_safe * 128
        for k in range(8):
          idx = base + (k * 16 + iota)
          cur = plsc.bitcast(plsc.load_gather(aggf, [idx]), jnp.bfloat16)
          val = plsc.bitcast(rows_ref[r, pl.ds(k * 16, 16)], jnp.bfloat16)
          mx = plsc.bitcast(jnp.maximum(cur, val), jnp.int32)
          plsc.store_scatter(aggf, [idx], mx)
        return 0
      lax.fori_loop(0, jnp.minimum(G, n - gb), row_body, 0)

    @pl.when(ng > 0)
    def _():
      issue(0, rows_a, sem_a)

    def pair_body(p, _):
      ga = 2 * p
      gb_ = 2 * p + 1

      @pl.when(gb_ < ng)
      def _():
        issue(gb_, rows_b, sem_b)
      drain(rows_a, sem_a)
      process(ga, rows_a)

      @pl.when(gb_ < ng)
      def _():
        @pl.when(gb_ + 1 < ng)
        def _():
          issue(gb_ + 1, rows_a, sem_a)
        drain(rows_b, sem_b)
        process(gb_, rows_b)
      return 0

    lax.fori_loop(0, (ng + 1) // 2, pair_body, 0)

  def do_chunk(c, buf, off_vec):
    cbase = c * DCH

    def comp_body(i, ov):
      d = buf[pl.ds(i * 16, 16)]
      m = (d >= lo) & (d < hi)
      pos = ov + plsc.cumsum(m.astype(jnp.int32)) - 1
      eid = cbase + i * 16 + iota
      plsc.store_scatter(meid, [pos], eid, mask=m)
      plsc.store_scatter(mld, [pos], d - lo, mask=m)
      return ov + plsc.all_reduce_population_count(m)

    off_vec = lax.fori_loop(0, DCH // 16, comp_body, off_vec)
    n = jnp.max(off_vec)
    do_flush = (n >= FLUSH) | (c >= NCHUNK - 1)

    @pl.when(do_flush)
    def _():
      flush(n)

    return jnp.where(jnp.full((16,), do_flush), 0, off_vec)

  def chunk_body(c, off_vec):
    pltpu.sync_copy(dst_hbm.at[pl.ds(c * DCH, DCH)], dstbuf)
    return do_chunk(c, dstbuf, off_vec)

  lax.fori_loop(0, NCHUNK, chunk_body, jnp.zeros((16,), jnp.int32))

  pltpu.sync_copy(aggf.at[pl.ds(0, ROWS_PER_W * 128)],
                  agg_hbm.at[pl.ds(wid * ROWS_PER_W * 128,
                                   ROWS_PER_W * 128)])


def _stage_d(h3, dst):
  mesh = plsc.VectorSubcoreMesh(core_axis_name="c", subcore_axis_name="s")
  f = pl.kernel(
      _segmax_body,
      out_type=jax.ShapeDtypeStruct((N_PAD * 128,), jnp.int32),
      mesh=mesh,
      scratch_types=[
          pltpu.VMEM((AGG_W,), jnp.int32),
          pltpu.VMEM((DCH,), jnp.int32),
          pltpu.VMEM((MBUF,), jnp.int32),
          pltpu.VMEM((MBUF,), jnp.int32),
          pltpu.VMEM((G, 128), jnp.int32),
          pltpu.VMEM((G, 128), jnp.int32),
          pltpu.SemaphoreType.DMA,
          pltpu.SemaphoreType.DMA,
      ],
      compiler_params=pltpu.CompilerParams(needs_layout_passes=False),
  )
  return f(h3, dst)


def _out_body(aggi_ref, wge_ref, wgo_ref, bg_ref, out_ref):
  v = aggi_ref[...]
  ae = jax.lax.bitcast_convert_type(v << 16, jnp.float32)
  ao = jax.lax.bitcast_convert_type(v & jnp.int32(-65536), jnp.float32)
  ae = jnp.where(ae > -1.0e29, ae, 0.0).astype(jnp.bfloat16)
  ao = jnp.where(ao > -1.0e29, ao, 0.0).astype(jnp.bfloat16)
  out_ref[...] = jnp.maximum(
      jnp.dot(ae, wge_ref[...], preferred_element_type=jnp.float32)
      + jnp.dot(ao, wgo_ref[...], preferred_element_type=jnp.float32)
      + bg_ref[...], 0.0)


def _stage_e(aggi, wge, wgo, bg):
  blk = 1000
  return pl.pallas_call(
      _out_body,
      grid=(N // blk,),
      in_specs=[
          pl.BlockSpec((blk, 128), lambda i: (i, 0)),
          pl.BlockSpec((128, 256), lambda i: (0, 0)),
          pl.BlockSpec((128, 256), lambda i: (0, 0)),
          pl.BlockSpec((1, 256), lambda i: (0, 0)),
      ],
      out_specs=pl.BlockSpec((blk, 256), lambda i: (i, 0)),
      out_shape=jax.ShapeDtypeStruct((N, 256), jnp.float32),
      compiler_params=pltpu.CompilerParams(
          dimension_semantics=("arbitrary",)),
  )(aggi, wge, wgo, bg)


def kernel(x, pos, edge_index, W1, b1, W2, b2, W3, b3, Wg, bg):
  src = edge_index[0].astype(jnp.int32)
  dst = edge_index[1].astype(jnp.int32)
  w1a = W1[:256]
  w1bp = jnp.zeros((8, DZ), jnp.float32).at[:3].set(W1[256:])
  posp = jnp.zeros((N, 8), jnp.float32).at[:, :3].set(pos)

  z, w = _stage_a(x, posp, w1a, w1bp, b1.reshape(1, DZ))
  zs, wd = _stage_b(z, w, src, dst)
  w3b = W3.astype(jnp.bfloat16)
  h3 = _stage_c(zs, wd, W2.astype(jnp.bfloat16), b2.reshape(1, 128),
                w3b[:, 0::2], b3[0::2].reshape(1, 128),
                w3b[:, 1::2], b3[1::2].reshape(1, 128))
  agg1d = _stage_d(h3, dst)
  aggi = agg1d.reshape(N_PAD, 128)
  return _stage_e(aggi, Wg[0::2].astype(jnp.bfloat16),
                  Wg[1::2].astype(jnp.bfloat16), bg.reshape(1, 256))
